# baseline (speedup 1.0000x reference)
import functools

import jax
import jax.numpy as jnp
import numpy as np
from jax import lax
from jax.experimental import pallas as pl
from jax.experimental.pallas import tpu as pltpu
from jax.experimental.pallas import tpu_sc as plsc

N = 10000
NP = 10240
E = 320000
EP = 331776
F = 128
D = 4

NC = 2
NS = 16

K = 96
EDGES_PER_TILE = EP // NS
NUM_CHUNKS = EDGES_PER_TILE // K
CPB = 8
NBLK = NUM_CHUNKS // CPB
ROWS_PER_TILE = NP // NS
NBG = 3
NBS = 2
AHEAD = 2
STEP = 6

_PERM = np.empty((F,), np.int32)
for _g in range(F // 32):
    for _t in range(32):
        _PERM[32 * _g + _t] = 32 * _g + (
            _t // 2 if _t % 2 == 0 else 16 + _t // 2)


def _mm_body(x_ref, w_ref, o_ref):
    o_ref[...] = lax.dot_general(
        x_ref[...], w_ref[...], (((1,), (1,)), ((), ())),
        preferred_element_type=jnp.float32,
        precision=lax.Precision.HIGHEST).astype(jnp.bfloat16)


def _matmul(x, w):
    blk = 1024
    return pl.pallas_call(
        _mm_body,
        grid=(NP // blk,),
        in_specs=[
            pl.BlockSpec((blk, F), lambda i: (i, 0)),
            pl.BlockSpec((F, F), lambda i: (0, 0)),
        ],
        out_specs=pl.BlockSpec((blk, F), lambda i: (i, 0)),
        out_shape=jax.ShapeDtypeStruct((NP, F), jnp.bfloat16),
    )(x, w)


def _sc_kernel(y, ef5, src4, dst4, bb):
    mesh = plsc.VectorSubcoreMesh(
        core_axis_name="c", subcore_axis_name="s",
        num_cores=NC, num_subcores=NS)

    @functools.partial(
        pl.kernel,
        out_type=jax.ShapeDtypeStruct((NP, D * F), jnp.float32),
        mesh=mesh,
        compiler_params=pltpu.CompilerParams(
            needs_layout_passes=False, use_tc_tiling_on_sc=False),
        scratch_types=[
            pltpu.VMEM_SHARED((NP, F), jnp.float32),
            pltpu.VMEM((2, CPB, K), jnp.int32),
            pltpu.VMEM((2, CPB, K), jnp.int32),
            pltpu.VMEM((2, CPB, K), jnp.float32),
            [pltpu.VMEM((K, F), jnp.bfloat16)] * NBG,
            [pltpu.VMEM((K, F), jnp.float32)] * NBS,
            [pltpu.SemaphoreType.DMA] * NBG,
            [pltpu.SemaphoreType.DMA] * NBS,
            pltpu.SemaphoreType.DMA,
        ],
    )
    def k(y_hbm, ef_hbm, src_hbm, dst_hbm, bb_hbm, out_hbm,
          acc_sh, srcs, dsts, es, rbf, rf32, gsems, ssems, bsem):
        c = lax.axis_index("c")
        s = lax.axis_index("s")
        row0 = s * ROWS_PER_TILE

        def gather_start(ci, gb):
            pb = lax.rem(lax.div(ci, CPB), 2)
            pltpu.make_async_copy(
                y_hbm.at[srcs.at[pb, lax.rem(ci, CPB)]], rbf[gb],
                gsems[gb]).start()

        def gather_wait(gb):
            pltpu.make_async_copy(y_hbm.at[srcs.at[0, 0]], rbf[gb],
                                  gsems[gb]).wait()

        def scatter_start(ci, sb):
            pb = lax.rem(lax.div(ci, CPB), 2)
            pltpu.make_async_copy(
                rf32[sb], acc_sh.at[dsts.at[pb, lax.rem(ci, CPB)]],
                ssems[sb]).start(add=True)

        def scatter_wait(sb):
            pltpu.make_async_copy(rf32[sb], acc_sh.at[dsts.at[0, 0]],
                                  ssems[sb]).wait()

        for pass_i in range(2):
            d = c * 2 + pass_i

            def block_descs(blk, pb):
                return (
                    pltpu.make_async_copy(src_hbm.at[s, blk], srcs.at[pb],
                                          bsem),
                    pltpu.make_async_copy(dst_hbm.at[s, blk], dsts.at[pb],
                                          bsem),
                    pltpu.make_async_copy(ef_hbm.at[d, s, blk], es.at[pb],
                                          bsem),
                )

            pltpu.sync_copy(bb_hbm.at[pl.ds(row0, ROWS_PER_TILE), :],
                            acc_sh.at[pl.ds(row0, ROWS_PER_TILE), :])
            plsc.subcore_barrier()

            for desc in block_descs(0, 0):
                desc.start()
            for desc in block_descs(0, 0):
                desc.wait()
            for b in range(AHEAD):
                gather_start(b, b % NBG)

            def process(ci, gb, sb):
                rem = lax.rem(ci, CPB)
                blk = lax.div(ci, CPB)
                pb = lax.rem(blk, 2)

                @pl.when(jnp.logical_and(rem == CPB - AHEAD, blk + 1 < NBLK))
                def _():
                    for desc in block_descs(blk + 1, 1 - pb):
                        desc.wait()

                nci = ci + AHEAD
                @pl.when(nci < NUM_CHUNKS)
                def _():
                    gather_start(nci, (gb + AHEAD) % NBG)

                @pl.when(ci - NBS >= 0)
                def _():
                    scatter_wait(sb)

                @pl.when(jnp.logical_and(rem == 1, blk + 1 < NBLK))
                def _():
                    for desc in block_descs(blk + 1, 1 - pb):
                        desc.start()

                gather_wait(gb)

                rbf_b = rbf[gb]
                rf32_b = rf32[sb]

                @plsc.parallel_loop(0, K, unroll=16)
                def _scale(r):
                    eb = plsc.load_gather(
                        es, [jnp.full((16,), pb, jnp.int32),
                             jnp.full((16,), rem, jnp.int32),
                             jnp.full((16,), r, jnp.int32)])
                    for g in range(F // 32):
                        v = rbf_b[r, pl.ds(g * 32, 32)]
                        lo, hi = plsc.unpack(
                            v, format=plsc.PackFormat.INTERLEAVED,
                            preferred_element_type=jnp.float32)
                        rf32_b[r, pl.ds(g * 32, 16)] = lo * eb
                        rf32_b[r, pl.ds(g * 32 + 16, 16)] = hi * eb

                scatter_start(ci, sb)

            main_end = (NUM_CHUNKS // STEP) * STEP

            @pl.loop(0, main_end, step=STEP)
            def _group(i):
                for b in range(STEP):
                    process(i + b, b % NBG, b % NBS)

            for ci in range(main_end, NUM_CHUNKS):
                process(ci, ci % NBG, ci % NBS)

            for ci in range(NUM_CHUNKS - NBS, NUM_CHUNKS):
                scatter_wait(ci % NBS)
            plsc.subcore_barrier()
            pltpu.sync_copy(
                acc_sh.at[pl.ds(row0, ROWS_PER_TILE), :],
                out_hbm.at[pl.ds(row0, ROWS_PER_TILE), pl.ds(d * F, F)])
            plsc.subcore_barrier()

    return k(y, ef5, src4, dst4, bb)


def kernel(node_feat, edge_feat, edge_index, W, b):
    x = jnp.pad(node_feat, ((0, NP - N), (0, 0)))
    y = _matmul(x, W)[:, jnp.asarray(_PERM)]
    pad = EP - E
    ef5 = jnp.pad(edge_feat.T, ((0, 0), (0, pad))).reshape(
        D, NS, NBLK, CPB, K)
    src4 = jnp.pad(edge_index[0], (0, pad)).reshape(NS, NBLK, CPB, K)
    dst4 = jnp.pad(edge_index[1], (0, pad),
                   constant_values=N).reshape(NS, NBLK, CPB, K)
    bb = jnp.broadcast_to(b, (NP, F))
    return _sc_kernel(y, ef5, src4, dst4, bb)[:N]


# speedup vs baseline: 1.0002x; 1.0002x over previous
import functools

import jax
import jax.numpy as jnp
import numpy as np
from jax import lax
from jax.experimental import pallas as pl
from jax.experimental.pallas import tpu as pltpu
from jax.experimental.pallas import tpu_sc as plsc

N = 10000
NP = 10240
E = 320000
EP = 331776
F = 128
D = 4

NC = 2
NS = 16

K = 96
EDGES_PER_TILE = EP // NS
NUM_CHUNKS = EDGES_PER_TILE // K
CPB = 8
NBLK = NUM_CHUNKS // CPB
ROWS_PER_TILE = NP // NS
NBG = 3
NBS = 2
AHEAD = 2
STEP = 6

_PERM = np.empty((F,), np.int32)
for _g in range(F // 32):
    for _t in range(32):
        _PERM[32 * _g + _t] = 32 * _g + (
            _t // 2 if _t % 2 == 0 else 16 + _t // 2)


def _mm_body(x_ref, w_ref, o_ref):
    o_ref[...] = lax.dot_general(
        x_ref[...], w_ref[...], (((1,), (1,)), ((), ())),
        preferred_element_type=jnp.float32,
        precision=lax.Precision.HIGHEST).astype(jnp.bfloat16)


def _matmul(x, w):
    blk = 1024
    return pl.pallas_call(
        _mm_body,
        grid=(NP // blk,),
        in_specs=[
            pl.BlockSpec((blk, F), lambda i: (i, 0)),
            pl.BlockSpec((F, F), lambda i: (0, 0)),
        ],
        out_specs=pl.BlockSpec((blk, F), lambda i: (i, 0)),
        out_shape=jax.ShapeDtypeStruct((NP, F), jnp.bfloat16),
    )(x, w)


def _sc_kernel(y, ef5, src4, dst4, bb):
    mesh = plsc.VectorSubcoreMesh(
        core_axis_name="c", subcore_axis_name="s",
        num_cores=NC, num_subcores=NS)

    @functools.partial(
        pl.kernel,
        out_type=jax.ShapeDtypeStruct((NP, D * F), jnp.float32),
        mesh=mesh,
        compiler_params=pltpu.CompilerParams(
            needs_layout_passes=False, use_tc_tiling_on_sc=False),
        scratch_types=[
            pltpu.VMEM_SHARED((NP, F), jnp.float32),
            pltpu.VMEM((2, CPB, K), jnp.int32),
            pltpu.VMEM((2, CPB, K), jnp.int32),
            pltpu.VMEM((2, CPB, K), jnp.float32),
            [pltpu.VMEM((K, F), jnp.bfloat16)] * NBG,
            [pltpu.VMEM((K, F), jnp.float32)] * NBS,
            [pltpu.SemaphoreType.DMA] * NBG,
            [pltpu.SemaphoreType.DMA] * NBS,
            pltpu.SemaphoreType.DMA,
        ],
    )
    def k(y_hbm, ef_hbm, src_hbm, dst_hbm, bb_hbm, out_hbm,
          acc_sh, srcs, dsts, es, rbf, rf32, gsems, ssems, bsem):
        c = lax.axis_index("c")
        s = lax.axis_index("s")
        row0 = s * ROWS_PER_TILE

        def gather_start(ci, gb):
            pb = lax.rem(lax.div(ci, CPB), 2)
            pltpu.make_async_copy(
                y_hbm.at[srcs.at[pb, lax.rem(ci, CPB)]], rbf[gb],
                gsems[gb]).start()

        def gather_wait(gb):
            pltpu.make_async_copy(y_hbm.at[srcs.at[0, 0]], rbf[gb],
                                  gsems[gb]).wait()

        def scatter_start(ci, sb):
            pb = lax.rem(lax.div(ci, CPB), 2)
            pltpu.make_async_copy(
                rf32[sb], acc_sh.at[dsts.at[pb, lax.rem(ci, CPB)]],
                ssems[sb]).start(add=True)

        def scatter_wait(sb):
            pltpu.make_async_copy(rf32[sb], acc_sh.at[dsts.at[0, 0]],
                                  ssems[sb]).wait()

        for pass_i in range(2):
            d = c * 2 + pass_i

            def block_descs(blk, pb):
                return (
                    pltpu.make_async_copy(src_hbm.at[s, blk], srcs.at[pb],
                                          bsem),
                    pltpu.make_async_copy(dst_hbm.at[s, blk], dsts.at[pb],
                                          bsem),
                    pltpu.make_async_copy(ef_hbm.at[d, s, blk], es.at[pb],
                                          bsem),
                )

            pltpu.sync_copy(bb_hbm.at[pl.ds(row0, ROWS_PER_TILE), :],
                            acc_sh.at[pl.ds(row0, ROWS_PER_TILE), :])
            plsc.subcore_barrier()

            for desc in block_descs(0, 0):
                desc.start()
            for desc in block_descs(0, 0):
                desc.wait()
            for b in range(AHEAD):
                gather_start(b, b % NBG)

            def process(ci, gb, sb):
                rem = lax.rem(ci, CPB)
                blk = lax.div(ci, CPB)
                pb = lax.rem(blk, 2)

                @pl.when(jnp.logical_and(rem == CPB - AHEAD, blk + 1 < NBLK))
                def _():
                    for desc in block_descs(blk + 1, 1 - pb):
                        desc.wait()

                nci = ci + AHEAD
                @pl.when(nci < NUM_CHUNKS)
                def _():
                    gather_start(nci, (gb + AHEAD) % NBG)

                @pl.when(ci - NBS >= 0)
                def _():
                    scatter_wait(sb)

                @pl.when(jnp.logical_and(rem == 1, blk + 1 < NBLK))
                def _():
                    for desc in block_descs(blk + 1, 1 - pb):
                        desc.start()

                gather_wait(gb)

                rbf_b = rbf[gb]
                rf32_b = rf32[sb]

                @plsc.parallel_loop(0, K, unroll=16)
                def _scale(r):
                    eb = plsc.load_gather(
                        es, [jnp.full((16,), pb, jnp.int32),
                             jnp.full((16,), rem, jnp.int32),
                             jnp.full((16,), r, jnp.int32)])
                    for g in range(F // 32):
                        v = rbf_b[r, pl.ds(g * 32, 32)]
                        lo, hi = plsc.unpack(
                            v, format=plsc.PackFormat.INTERLEAVED,
                            preferred_element_type=jnp.float32)
                        rf32_b[r, pl.ds(g * 32, 16)] = lo * eb
                        rf32_b[r, pl.ds(g * 32 + 16, 16)] = hi * eb

                scatter_start(ci, sb)

            main_end = (NUM_CHUNKS // STEP) * STEP

            @pl.loop(0, main_end, step=STEP)
            def _group(i):
                for b in range(STEP):
                    process(i + b, b % NBG, b % NBS)

            for ci in range(main_end, NUM_CHUNKS):
                process(ci, ci % NBG, ci % NBS)

            for ci in range(NUM_CHUNKS - NBS, NUM_CHUNKS):
                scatter_wait(ci % NBS)
            plsc.subcore_barrier()
            pltpu.sync_copy(
                acc_sh.at[pl.ds(row0, ROWS_PER_TILE), :],
                out_hbm.at[pl.ds(row0, ROWS_PER_TILE), pl.ds(d * F, F)])
            plsc.subcore_barrier()

    return k(y, ef5, src4, dst4, bb)


def kernel(node_feat, edge_feat, edge_index, W, b):
    x = jnp.pad(node_feat, ((0, NP - N), (0, 0)))
    y = _matmul(x, W)[:, jnp.asarray(_PERM)]
    pad = EP - E
    ef5 = jnp.pad(edge_feat.T, ((0, 0), (0, pad))).reshape(
        D, NS, NBLK, CPB, K)
    src4 = jnp.pad(edge_index[0], (0, pad)).reshape(NS, NBLK, CPB, K)
    junk = N + (jnp.arange(pad, dtype=jnp.int32) % (NP - N))
    dst4 = jnp.concatenate([edge_index[1], junk]).reshape(NS, NBLK, CPB, K)
    bb = jnp.broadcast_to(b, (NP, F))
    return _sc_kernel(y, ef5, src4, dst4, bb)[:N]


# speedup vs baseline: 2.6287x; 2.6281x over previous
import functools

import jax
import jax.numpy as jnp
import numpy as np
from jax import lax
from jax.experimental import pallas as pl
from jax.experimental.pallas import tpu as pltpu
from jax.experimental.pallas import tpu_sc as plsc

N = 10000
NP = 10240
E = 320000
F = 128
D = 4

NC = 2
NS = 16

K = 80
EDGES_PER_TILE = E // NS
NUM_CHUNKS = EDGES_PER_TILE // K
CPB = 5
NBLK = NUM_CHUNKS // CPB
ROWS_PER_TILE = NP // NS
NBG = 4
NBS = 2
AHEAD = 3
STEP = 4

_PERM = np.empty((F,), np.int32)
for _g in range(F // 32):
    for _t in range(32):
        _PERM[32 * _g + _t] = 32 * _g + (
            _t // 2 if _t % 2 == 0 else 16 + _t // 2)


def _mm_body(x_ref, w_ref, o_ref):
    o_ref[...] = lax.dot_general(
        x_ref[...], w_ref[...], (((1,), (1,)), ((), ())),
        preferred_element_type=jnp.float32,
        precision=lax.Precision.HIGHEST).astype(jnp.bfloat16)


def _matmul(x, w):
    blk = 1024
    return pl.pallas_call(
        _mm_body,
        grid=(NP // blk,),
        in_specs=[
            pl.BlockSpec((blk, F), lambda i: (i, 0)),
            pl.BlockSpec((F, F), lambda i: (0, 0)),
        ],
        out_specs=pl.BlockSpec((blk, F), lambda i: (i, 0)),
        out_shape=jax.ShapeDtypeStruct((NP, F), jnp.bfloat16),
    )(x, w)


def _sc_kernel(y, ef5, src4, dst4, bb):
    mesh = plsc.VectorSubcoreMesh(
        core_axis_name="c", subcore_axis_name="s",
        num_cores=NC, num_subcores=NS)

    @functools.partial(
        pl.kernel,
        out_type=jax.ShapeDtypeStruct((NP, D * F), jnp.float32),
        mesh=mesh,
        compiler_params=pltpu.CompilerParams(
            needs_layout_passes=False, use_tc_tiling_on_sc=False),
        scratch_types=[
            pltpu.VMEM_SHARED((NP, F), jnp.float32),
            pltpu.VMEM((2, CPB, K), jnp.int32),
            pltpu.VMEM((2, CPB, K), jnp.int32),
            pltpu.VMEM((2, CPB, K), jnp.float32),
            [pltpu.VMEM((K, F), jnp.bfloat16)] * NBG,
            [pltpu.VMEM((K, F), jnp.float32)] * NBS,
            [pltpu.SemaphoreType.DMA] * NBG,
            [pltpu.SemaphoreType.DMA] * NBS,
            pltpu.SemaphoreType.DMA,
        ],
    )
    def k(y_hbm, ef_hbm, src_hbm, dst_hbm, bb_hbm, out_hbm,
          acc_sh, srcs, dsts, es, rbf, rf32, gsems, ssems, bsem):
        c = lax.axis_index("c")
        s = lax.axis_index("s")
        row0 = s * ROWS_PER_TILE

        def gather_start(ci, gb):
            pb = lax.rem(lax.div(ci, CPB), 2)
            pltpu.make_async_copy(
                y_hbm.at[srcs.at[pb, lax.rem(ci, CPB)]], rbf[gb],
                gsems[gb]).start()

        def gather_wait(gb):
            pltpu.make_async_copy(y_hbm.at[srcs.at[0, 0]], rbf[gb],
                                  gsems[gb]).wait()

        def scatter_start(ci, sb):
            pb = lax.rem(lax.div(ci, CPB), 2)
            pltpu.make_async_copy(
                rf32[sb], acc_sh.at[dsts.at[pb, lax.rem(ci, CPB)]],
                ssems[sb]).start(add=True)

        def scatter_wait(sb):
            pltpu.make_async_copy(rf32[sb], acc_sh.at[dsts.at[0, 0]],
                                  ssems[sb]).wait()

        for pass_i in range(2):
            d = c * 2 + pass_i

            def block_descs(blk, pb):
                return (
                    pltpu.make_async_copy(src_hbm.at[s, blk], srcs.at[pb],
                                          bsem),
                    pltpu.make_async_copy(dst_hbm.at[s, blk], dsts.at[pb],
                                          bsem),
                    pltpu.make_async_copy(ef_hbm.at[d, s, blk], es.at[pb],
                                          bsem),
                )

            pltpu.sync_copy(bb_hbm.at[pl.ds(row0, ROWS_PER_TILE), :],
                            acc_sh.at[pl.ds(row0, ROWS_PER_TILE), :])
            plsc.subcore_barrier()

            for desc in block_descs(0, 0):
                desc.start()
            for desc in block_descs(0, 0):
                desc.wait()
            for b in range(AHEAD):
                gather_start(b, b % NBG)

            def process(ci, gb, sb):
                rem = lax.rem(ci, CPB)
                blk = lax.div(ci, CPB)
                pb = lax.rem(blk, 2)

                @pl.when(jnp.logical_and(rem == CPB - AHEAD, blk + 1 < NBLK))
                def _():
                    for desc in block_descs(blk + 1, 1 - pb):
                        desc.wait()

                nci = ci + AHEAD
                @pl.when(nci < NUM_CHUNKS)
                def _():
                    gather_start(nci, (gb + AHEAD) % NBG)

                @pl.when(ci - NBS >= 0)
                def _():
                    scatter_wait(sb)

                @pl.when(jnp.logical_and(rem == 1, blk + 1 < NBLK))
                def _():
                    for desc in block_descs(blk + 1, 1 - pb):
                        desc.start()

                gather_wait(gb)

                rbf_b = rbf[gb]
                rf32_b = rf32[sb]

                @plsc.parallel_loop(0, K, unroll=16)
                def _scale(r):
                    eb = plsc.load_gather(
                        es, [jnp.full((16,), pb, jnp.int32),
                             jnp.full((16,), rem, jnp.int32),
                             jnp.full((16,), r, jnp.int32)])
                    for g in range(F // 32):
                        v = rbf_b[r, pl.ds(g * 32, 32)]
                        lo, hi = plsc.unpack(
                            v, format=plsc.PackFormat.INTERLEAVED,
                            preferred_element_type=jnp.float32)
                        rf32_b[r, pl.ds(g * 32, 16)] = lo * eb
                        rf32_b[r, pl.ds(g * 32 + 16, 16)] = hi * eb

                scatter_start(ci, sb)

            main_end = (NUM_CHUNKS // STEP) * STEP

            @pl.loop(0, main_end, step=STEP)
            def _group(i):
                for b in range(STEP):
                    process(i + b, b % NBG, b % NBS)

            for ci in range(main_end, NUM_CHUNKS):
                process(ci, ci % NBG, ci % NBS)

            for ci in range(NUM_CHUNKS - NBS, NUM_CHUNKS):
                scatter_wait(ci % NBS)
            plsc.subcore_barrier()
            pltpu.sync_copy(
                acc_sh.at[pl.ds(row0, ROWS_PER_TILE), :],
                out_hbm.at[pl.ds(row0, ROWS_PER_TILE), pl.ds(d * F, F)])
            plsc.subcore_barrier()

    return k(y, ef5, src4, dst4, bb)


def kernel(node_feat, edge_feat, edge_index, W, b):
    x = jnp.pad(node_feat, ((0, NP - N), (0, 0)))
    y = _matmul(x, W)[:, jnp.asarray(_PERM)]
    ef5 = edge_feat.T.reshape(D, NS, NBLK, CPB, K)
    src4 = edge_index[0].reshape(NS, NBLK, CPB, K)
    dst4 = edge_index[1].reshape(NS, NBLK, CPB, K)
    bb = jnp.broadcast_to(b, (NP, F))
    return _sc_kernel(y, ef5, src4, dst4, bb)[:N]


# speedup vs baseline: 2.7822x; 1.0584x over previous
import functools

import jax
import jax.numpy as jnp
import numpy as np
from jax import lax
from jax.experimental import pallas as pl
from jax.experimental.pallas import tpu as pltpu
from jax.experimental.pallas import tpu_sc as plsc

N = 10000
NP = 10240
E = 320000
F = 128
D = 4

NC = 2
NS = 16

K = 80
EDGES_PER_TILE = E // NS
NUM_CHUNKS = EDGES_PER_TILE // K
CPB = 5
NBLK = NUM_CHUNKS // CPB
ROWS_PER_TILE = N // NS
NBG = 3
NBS = 3
AHEAD = 2
STEP = 3

_PERM = np.empty((F,), np.int32)
for _g in range(F // 32):
    for _t in range(32):
        _PERM[32 * _g + _t] = 32 * _g + (
            _t // 2 if _t % 2 == 0 else 16 + _t // 2)


def _mm_body(x_ref, w_ref, o_ref):
    o_ref[...] = lax.dot_general(
        x_ref[...], w_ref[...], (((1,), (1,)), ((), ())),
        preferred_element_type=jnp.float32,
        precision=lax.Precision.HIGHEST).astype(jnp.bfloat16)


def _matmul(x, w):
    blk = 1024
    return pl.pallas_call(
        _mm_body,
        grid=(NP // blk,),
        in_specs=[
            pl.BlockSpec((blk, F), lambda i: (i, 0)),
            pl.BlockSpec((F, F), lambda i: (0, 0)),
        ],
        out_specs=pl.BlockSpec((blk, F), lambda i: (i, 0)),
        out_shape=jax.ShapeDtypeStruct((NP, F), jnp.bfloat16),
    )(x, w)


def _sc_kernel(y, ef5, src4, dst4, bb):
    mesh = plsc.VectorSubcoreMesh(
        core_axis_name="c", subcore_axis_name="s",
        num_cores=NC, num_subcores=NS)

    @functools.partial(
        pl.kernel,
        out_type=jax.ShapeDtypeStruct((N, D * F), jnp.float32),
        mesh=mesh,
        compiler_params=pltpu.CompilerParams(
            needs_layout_passes=False, use_tc_tiling_on_sc=False),
        scratch_types=[
            pltpu.VMEM_SHARED((N, F), jnp.float32),
            pltpu.VMEM((2, CPB, K), jnp.int32),
            pltpu.VMEM((2, CPB, K), jnp.int32),
            pltpu.VMEM((2, CPB, K), jnp.float32),
            [pltpu.VMEM((K, F), jnp.bfloat16)] * NBG,
            [pltpu.VMEM((K, F), jnp.float32)] * NBS,
            [pltpu.SemaphoreType.DMA] * NBG,
            [pltpu.SemaphoreType.DMA] * NBS,
            pltpu.SemaphoreType.DMA,
        ],
    )
    def k(y_hbm, ef_hbm, src_hbm, dst_hbm, bb_hbm, out_hbm,
          acc_sh, srcs, dsts, es, rbf, rf32, gsems, ssems, bsem):
        c = lax.axis_index("c")
        s = lax.axis_index("s")
        row0 = s * ROWS_PER_TILE

        def gather_start(ci, gb):
            pb = lax.rem(lax.div(ci, CPB), 2)
            pltpu.make_async_copy(
                y_hbm.at[srcs.at[pb, lax.rem(ci, CPB)]], rbf[gb],
                gsems[gb]).start()

        def gather_wait(gb):
            pltpu.make_async_copy(y_hbm.at[srcs.at[0, 0]], rbf[gb],
                                  gsems[gb]).wait()

        def scatter_start(ci, sb):
            pb = lax.rem(lax.div(ci, CPB), 2)
            pltpu.make_async_copy(
                rf32[sb], acc_sh.at[dsts.at[pb, lax.rem(ci, CPB)]],
                ssems[sb]).start(add=True)

        def scatter_wait(sb):
            pltpu.make_async_copy(rf32[sb], acc_sh.at[dsts.at[0, 0]],
                                  ssems[sb]).wait()

        for pass_i in range(2):
            d = c * 2 + pass_i

            def block_descs(blk, pb):
                return (
                    pltpu.make_async_copy(src_hbm.at[s, blk], srcs.at[pb],
                                          bsem),
                    pltpu.make_async_copy(dst_hbm.at[s, blk], dsts.at[pb],
                                          bsem),
                    pltpu.make_async_copy(ef_hbm.at[d, s, blk], es.at[pb],
                                          bsem),
                )

            pltpu.sync_copy(bb_hbm.at[pl.ds(row0, ROWS_PER_TILE), :],
                            acc_sh.at[pl.ds(row0, ROWS_PER_TILE), :])
            plsc.subcore_barrier()

            for desc in block_descs(0, 0):
                desc.start()
            for desc in block_descs(0, 0):
                desc.wait()
            for b in range(AHEAD):
                gather_start(b, b % NBG)

            def process(ci, gb, sb):
                rem = lax.rem(ci, CPB)
                blk = lax.div(ci, CPB)
                pb = lax.rem(blk, 2)

                @pl.when(jnp.logical_and(rem == CPB - AHEAD, blk + 1 < NBLK))
                def _():
                    for desc in block_descs(blk + 1, 1 - pb):
                        desc.wait()

                nci = ci + AHEAD
                @pl.when(nci < NUM_CHUNKS)
                def _():
                    gather_start(nci, (gb + AHEAD) % NBG)

                @pl.when(ci - NBS >= 0)
                def _():
                    scatter_wait(sb)

                @pl.when(jnp.logical_and(rem == 1, blk + 1 < NBLK))
                def _():
                    for desc in block_descs(blk + 1, 1 - pb):
                        desc.start()

                gather_wait(gb)

                rbf_b = rbf[gb]
                rf32_b = rf32[sb]

                @plsc.parallel_loop(0, K, unroll=16)
                def _scale(r):
                    eb = plsc.load_gather(
                        es, [jnp.full((16,), pb, jnp.int32),
                             jnp.full((16,), rem, jnp.int32),
                             jnp.full((16,), r, jnp.int32)])
                    for g in range(F // 32):
                        v = rbf_b[r, pl.ds(g * 32, 32)]
                        lo, hi = plsc.unpack(
                            v, format=plsc.PackFormat.INTERLEAVED,
                            preferred_element_type=jnp.float32)
                        rf32_b[r, pl.ds(g * 32, 16)] = lo * eb
                        rf32_b[r, pl.ds(g * 32 + 16, 16)] = hi * eb

                scatter_start(ci, sb)

            main_end = (NUM_CHUNKS // STEP) * STEP

            @pl.loop(0, main_end, step=STEP)
            def _group(i):
                for b in range(STEP):
                    process(i + b, b % NBG, b % NBS)

            for ci in range(main_end, NUM_CHUNKS):
                process(ci, ci % NBG, ci % NBS)

            for ci in range(NUM_CHUNKS - NBS, NUM_CHUNKS):
                scatter_wait(ci % NBS)
            plsc.subcore_barrier()
            pltpu.sync_copy(
                acc_sh.at[pl.ds(row0, ROWS_PER_TILE), :],
                out_hbm.at[pl.ds(row0, ROWS_PER_TILE), pl.ds(d * F, F)])
            plsc.subcore_barrier()

    return k(y, ef5, src4, dst4, bb)


def kernel(node_feat, edge_feat, edge_index, W, b):
    x = jnp.pad(node_feat, ((0, NP - N), (0, 0)))
    y = _matmul(x, W[jnp.asarray(_PERM)])
    ef5 = edge_feat.T.reshape(D, NS, NBLK, CPB, K)
    src4 = edge_index[0].reshape(NS, NBLK, CPB, K)
    dst4 = edge_index[1].reshape(NS, NBLK, CPB, K)
    bb = jnp.broadcast_to(b, (N, F))
    return _sc_kernel(y, ef5, src4, dst4, bb)
